# baseline (speedup 1.0000x reference)
import functools

import jax
import jax.numpy as jnp
from jax import lax
from jax.experimental import pallas as pl
from jax.experimental.pallas import tpu as pltpu


def _conv_stats_kernel(x_ref, w_ref, y_ref, sum_ref, sq_ref, xp_ref,
                       slab_ref, *, nbp, c2, W, HW, K, G):
    j = pl.program_id(0)

    @pl.when(j == 0)
    def _init():
        sum_ref[...] = jnp.zeros_like(sum_ref)
        sq_ref[...] = jnp.zeros_like(sq_ref)
        xp_ref[:, 3:4, :] = jnp.zeros_like(xp_ref[:, 3:4, :])
        xp_ref[:, 7:8, :] = jnp.zeros_like(xp_ref[:, 7:8, :])
        xp_ref[:, :, 0:G] = jnp.zeros_like(xp_ref[:, :, 0:G])
        xp_ref[:, :, G + HW:] = jnp.zeros_like(xp_ref[:, :, G + HW:])

    xp_ref[:, 0:3, G:G + HW] = x_ref[:, 0]
    xp_ref[:, 4:7, G:G + HW] = x_ref[:, 1]

    col = lax.broadcasted_iota(jnp.int32, (1, 1, HW), 2) % W
    mask_l = (col != 0).astype(jnp.float32)
    mask_r = (col != (W - 1)).astype(jnp.float32)

    for kh in range(3):
        for kw in range(3):
            start = G + (kh - 1) * W + (kw - 1)
            v = xp_ref[:, :, start:start + HW]
            if kw == 0:
                v = v * mask_l
            elif kw == 2:
                v = v * mask_r
            t = kh * 3 + kw
            slab_ref[:, 8 * t:8 * t + 8, :] = v

    w_b = jnp.broadcast_to(w_ref[...][None], (nbp, c2, K))
    acc = lax.dot_general(
        w_b, slab_ref[...],
        dimension_numbers=(((2,), (1,)), ((0,), (0,))),
        preferred_element_type=jnp.float32)

    sum_ref[...] += jnp.sum(jnp.sum(acc, axis=2, keepdims=True), axis=0)
    sq_ref[...] += jnp.sum(jnp.sum(acc * acc, axis=2, keepdims=True), axis=0)
    y_ref[...] = acc.astype(jnp.bfloat16)


def _affine_kernel(y_ref, sum_ref, sq_ref, g_ref, b_ref, o_ref,
                   *, m_total, eps, c_out):
    inv_m = 1.0 / float(m_total)
    s = sum_ref[0:c_out] + sum_ref[c_out:2 * c_out]
    q = sq_ref[0:c_out] + sq_ref[c_out:2 * c_out]
    mean = s * inv_m
    var = jnp.maximum(q * inv_m - mean * mean, 0.0)
    inv_std = lax.rsqrt(var + eps)
    scale = g_ref[...] * inv_std
    shift = b_ref[...] - mean * scale
    c = scale.shape[0]
    scale2 = jnp.concatenate([scale, scale], axis=0)
    shift2 = jnp.concatenate([shift, shift], axis=0)
    y = y_ref[...].astype(jnp.float32)
    o = y * scale2[None] + shift2[None]
    nbp2 = o.shape[0]
    for p in range(nbp2):
        o_ref[2 * p] = o[p, 0:c]
        o_ref[2 * p + 1] = o[p, c:2 * c]


def _round_up(v, m):
    return (v + m - 1) // m * m


def kernel(x_nchw, conv_w_oihw, gamma, beta):
    eps = 1e-5
    N, C_in, H, W = x_nchw.shape
    C_out = conv_w_oihw.shape[0]
    HW = H * W
    K = 72
    Lx = (H + 2) * W + 2
    Lx_pad = _round_up(Lx, 128)
    vmem_limit = ((64 << 20) * 3) // 4

    NP = N // 2
    c2 = 2 * C_out
    G = 128
    nbp = 32
    while NP % nbp != 0:
        nbp //= 2
    steps = NP // nbp
    nbp2 = 64
    while NP % nbp2 != 0:
        nbp2 //= 2
    steps2 = NP // nbp2

    x_in = x_nchw.astype(jnp.float32).reshape(NP, 2, C_in, HW)

    w_t = jnp.transpose(conv_w_oihw.astype(jnp.float32), (0, 2, 3, 1))
    w_t = w_t.reshape(C_out, 9, C_in)
    w_lo = jnp.pad(w_t, ((0, 0), (0, 0), (0, 5))).reshape(C_out, K)
    w_hi = jnp.pad(w_t, ((0, 0), (0, 0), (4, 1))).reshape(C_out, K)
    w2 = jnp.concatenate([w_lo, w_hi], axis=0)
    g2 = gamma.reshape(C_out, 1).astype(jnp.float32)
    b2 = beta.reshape(C_out, 1).astype(jnp.float32)

    conv_stats = functools.partial(_conv_stats_kernel, nbp=nbp, c2=c2,
                                   W=W, HW=HW, K=K, G=G)
    y16, sums, sqs = pl.pallas_call(
        conv_stats,
        out_shape=(jax.ShapeDtypeStruct((NP, c2, HW), jnp.bfloat16),
                   jax.ShapeDtypeStruct((c2, 1), jnp.float32),
                   jax.ShapeDtypeStruct((c2, 1), jnp.float32)),
        grid=(steps,),
        in_specs=[
            pl.BlockSpec((nbp, 2, C_in, HW), lambda j: (j, 0, 0, 0)),
            pl.BlockSpec((c2, K), lambda j: (0, 0)),
        ],
        out_specs=(pl.BlockSpec((nbp, c2, HW), lambda j: (j, 0, 0)),
                   pl.BlockSpec((c2, 1), lambda j: (0, 0)),
                   pl.BlockSpec((c2, 1), lambda j: (0, 0))),
        scratch_shapes=[pltpu.VMEM((nbp, 8, 2 * G + HW), jnp.float32),
                        pltpu.VMEM((nbp, K, HW), jnp.float32)],
        compiler_params=pltpu.CompilerParams(
            dimension_semantics=("arbitrary",),
            vmem_limit_bytes=vmem_limit),
    )(x_in, w2)

    aff = functools.partial(_affine_kernel, m_total=N * H * W, eps=eps,
                            c_out=C_out)
    out_flat = pl.pallas_call(
        aff,
        out_shape=jax.ShapeDtypeStruct((N, C_out, HW), jnp.float32),
        grid=(steps2,),
        in_specs=[
            pl.BlockSpec((nbp2, c2, HW), lambda j: (j, 0, 0)),
            pl.BlockSpec((c2, 1), lambda j: (0, 0)),
            pl.BlockSpec((c2, 1), lambda j: (0, 0)),
            pl.BlockSpec((C_out, 1), lambda j: (0, 0)),
            pl.BlockSpec((C_out, 1), lambda j: (0, 0)),
        ],
        out_specs=pl.BlockSpec((2 * nbp2, C_out, HW), lambda j: (j, 0, 0)),
        compiler_params=pltpu.CompilerParams(
            dimension_semantics=("arbitrary",),
            vmem_limit_bytes=vmem_limit),
    )(y16, sums, sqs, g2, b2)

    return out_flat.reshape(N, C_out, H, W)


# speedup vs baseline: 1.1257x; 1.1257x over previous
import functools

import jax
import jax.numpy as jnp
from jax import lax
from jax.experimental import pallas as pl
from jax.experimental.pallas import tpu as pltpu


def _conv_stats_kernel(x_ref, w_ref, y_ref, sum_ref, sq_ref, xp_ref,
                       slab_ref, *, nbp, c2, W, HW, K, G):
    j = pl.program_id(0)

    @pl.when(j == 0)
    def _init():
        sum_ref[...] = jnp.zeros_like(sum_ref)
        sq_ref[...] = jnp.zeros_like(sq_ref)
        xp_ref[:, 3:4, :] = jnp.zeros_like(xp_ref[:, 3:4, :])
        xp_ref[:, 7:8, :] = jnp.zeros_like(xp_ref[:, 7:8, :])

    xp_ref[:, 0:3, :] = x_ref[:, 0]
    xp_ref[:, 4:7, :] = x_ref[:, 1]

    col = lax.broadcasted_iota(jnp.int32, (1, 1, HW), 2) % W
    mask_l = (col != 0).astype(jnp.float32)
    mask_r = (col != (W - 1)).astype(jnp.float32)

    for kh in range(3):
        for kw in range(3):
            start = kh * W + kw
            v = xp_ref[:, :, start:start + HW]
            if kw == 0:
                v = v * mask_l
            elif kw == 2:
                v = v * mask_r
            t = kh * 3 + kw
            slab_ref[:, 8 * t:8 * t + 8, :] = v

    w_b = jnp.broadcast_to(w_ref[...][None], (nbp, c2, K))
    acc = lax.dot_general(
        w_b, slab_ref[...],
        dimension_numbers=(((2,), (1,)), ((0,), (0,))),
        preferred_element_type=jnp.float32)

    sum_ref[...] += jnp.sum(jnp.sum(acc, axis=2, keepdims=True), axis=0)
    sq_ref[...] += jnp.sum(jnp.sum(acc * acc, axis=2, keepdims=True), axis=0)
    y_ref[...] = acc.astype(jnp.bfloat16)


def _affine_kernel(y_ref, sum_ref, sq_ref, g_ref, b_ref, o_ref,
                   *, m_total, eps, c_out):
    inv_m = 1.0 / float(m_total)
    s = sum_ref[0:c_out] + sum_ref[c_out:2 * c_out]
    q = sq_ref[0:c_out] + sq_ref[c_out:2 * c_out]
    mean = s * inv_m
    var = jnp.maximum(q * inv_m - mean * mean, 0.0)
    inv_std = lax.rsqrt(var + eps)
    scale = g_ref[...] * inv_std
    shift = b_ref[...] - mean * scale
    c = scale.shape[0]
    scale2 = jnp.concatenate([scale, scale], axis=0)
    shift2 = jnp.concatenate([shift, shift], axis=0)
    y = y_ref[...].astype(jnp.float32)
    o = y * scale2[None] + shift2[None]
    nbp2 = o.shape[0]
    for p in range(nbp2):
        o_ref[2 * p] = o[p, 0:c]
        o_ref[2 * p + 1] = o[p, c:2 * c]


def _round_up(v, m):
    return (v + m - 1) // m * m


def kernel(x_nchw, conv_w_oihw, gamma, beta):
    eps = 1e-5
    N, C_in, H, W = x_nchw.shape
    C_out = conv_w_oihw.shape[0]
    HW = H * W
    K = 72
    Lx = (H + 2) * W + 2
    Lx_pad = _round_up(Lx, 128)
    vmem_limit = ((64 << 20) * 3) // 4

    NP = N // 2
    c2 = 2 * C_out
    G = 128
    nbp = 32
    while NP % nbp != 0:
        nbp //= 2
    steps = NP // nbp
    nbp2 = 64
    while NP % nbp2 != 0:
        nbp2 //= 2
    steps2 = NP // nbp2

    xf = x_nchw.astype(jnp.float32)
    x_hp = jnp.pad(xf, ((0, 0), (0, 0), (1, 1), (0, 0)))
    x_flat = x_hp.reshape(N, C_in, (H + 2) * W)
    x_in = jnp.pad(x_flat, ((0, 0), (0, 0), (1, 1 + Lx_pad - Lx)))
    x_in = x_in.reshape(NP, 2, C_in, Lx_pad)

    w_t = jnp.transpose(conv_w_oihw.astype(jnp.float32), (0, 2, 3, 1))
    w_t = w_t.reshape(C_out, 9, C_in)
    w_lo = jnp.pad(w_t, ((0, 0), (0, 0), (0, 5))).reshape(C_out, K)
    w_hi = jnp.pad(w_t, ((0, 0), (0, 0), (4, 1))).reshape(C_out, K)
    w2 = jnp.concatenate([w_lo, w_hi], axis=0)
    g2 = gamma.reshape(C_out, 1).astype(jnp.float32)
    b2 = beta.reshape(C_out, 1).astype(jnp.float32)

    conv_stats = functools.partial(_conv_stats_kernel, nbp=nbp, c2=c2,
                                   W=W, HW=HW, K=K, G=G)
    y16, sums, sqs = pl.pallas_call(
        conv_stats,
        out_shape=(jax.ShapeDtypeStruct((NP, c2, HW), jnp.bfloat16),
                   jax.ShapeDtypeStruct((c2, 1), jnp.float32),
                   jax.ShapeDtypeStruct((c2, 1), jnp.float32)),
        grid=(steps,),
        in_specs=[
            pl.BlockSpec((nbp, 2, C_in, Lx_pad), lambda j: (j, 0, 0, 0)),
            pl.BlockSpec((c2, K), lambda j: (0, 0)),
        ],
        out_specs=(pl.BlockSpec((nbp, c2, HW), lambda j: (j, 0, 0)),
                   pl.BlockSpec((c2, 1), lambda j: (0, 0)),
                   pl.BlockSpec((c2, 1), lambda j: (0, 0))),
        scratch_shapes=[pltpu.VMEM((nbp, 8, Lx_pad), jnp.float32),
                        pltpu.VMEM((nbp, K, HW), jnp.float32)],
        compiler_params=pltpu.CompilerParams(
            dimension_semantics=("arbitrary",),
            vmem_limit_bytes=vmem_limit),
    )(x_in, w2)

    aff = functools.partial(_affine_kernel, m_total=N * H * W, eps=eps,
                            c_out=C_out)
    out_flat = pl.pallas_call(
        aff,
        out_shape=jax.ShapeDtypeStruct((N, C_out, HW), jnp.float32),
        grid=(steps2,),
        in_specs=[
            pl.BlockSpec((nbp2, c2, HW), lambda j: (j, 0, 0)),
            pl.BlockSpec((c2, 1), lambda j: (0, 0)),
            pl.BlockSpec((c2, 1), lambda j: (0, 0)),
            pl.BlockSpec((C_out, 1), lambda j: (0, 0)),
            pl.BlockSpec((C_out, 1), lambda j: (0, 0)),
        ],
        out_specs=pl.BlockSpec((2 * nbp2, C_out, HW), lambda j: (j, 0, 0)),
        compiler_params=pltpu.CompilerParams(
            dimension_semantics=("arbitrary",),
            vmem_limit_bytes=vmem_limit),
    )(y16, sums, sqs, g2, b2)

    return out_flat.reshape(N, C_out, H, W)


# speedup vs baseline: 1.1374x; 1.0104x over previous
import functools

import jax
import jax.numpy as jnp
from jax import lax
from jax.experimental import pallas as pl
from jax.experimental.pallas import tpu as pltpu


def _conv_stats_kernel(x_ref, w_ref, y_ref, sum_ref, sq_ref, xp_ref,
                       slab_ref, *, nbp, c2, W, HW, K, G):
    j = pl.program_id(0)

    @pl.when(j == 0)
    def _init():
        sum_ref[...] = jnp.zeros_like(sum_ref)
        sq_ref[...] = jnp.zeros_like(sq_ref)
        xp_ref[:, 3:4, :] = jnp.zeros_like(xp_ref[:, 3:4, :])
        xp_ref[:, 7:8, :] = jnp.zeros_like(xp_ref[:, 7:8, :])

    xp_ref[:, 0:3, :] = x_ref[:, 0]
    xp_ref[:, 4:7, :] = x_ref[:, 1]

    col = lax.broadcasted_iota(jnp.int32, (1, 1, HW), 2) % W
    mask_l = (col != 0).astype(jnp.float32)
    mask_r = (col != (W - 1)).astype(jnp.float32)

    for kh in range(3):
        for kw in range(3):
            start = kh * W + kw
            v = xp_ref[:, :, start:start + HW]
            if kw == 0:
                v = v * mask_l
            elif kw == 2:
                v = v * mask_r
            t = kh * 3 + kw
            slab_ref[:, 8 * t:8 * t + 8, :] = v

    w_b = jnp.broadcast_to(w_ref[...][None], (nbp, c2, K))
    acc = lax.dot_general(
        w_b, slab_ref[...],
        dimension_numbers=(((2,), (1,)), ((0,), (0,))),
        preferred_element_type=jnp.float32)

    sum_ref[...] += jnp.sum(jnp.sum(acc, axis=2, keepdims=True), axis=0)
    sq_ref[...] += jnp.sum(jnp.sum(acc * acc, axis=2, keepdims=True), axis=0)
    y_ref[...] = acc.astype(jnp.bfloat16)


def _affine_kernel(y_ref, sum_ref, sq_ref, g_ref, b_ref, o_ref,
                   *, m_total, eps, c_out):
    inv_m = 1.0 / float(m_total)
    s = sum_ref[0:c_out] + sum_ref[c_out:2 * c_out]
    q = sq_ref[0:c_out] + sq_ref[c_out:2 * c_out]
    mean = s * inv_m
    var = jnp.maximum(q * inv_m - mean * mean, 0.0)
    inv_std = lax.rsqrt(var + eps)
    scale = g_ref[...] * inv_std
    shift = b_ref[...] - mean * scale
    c = scale.shape[0]
    scale2 = jnp.concatenate([scale, scale], axis=0)
    shift2 = jnp.concatenate([shift, shift], axis=0)
    y = y_ref[...].astype(jnp.float32)
    o = y * scale2[None] + shift2[None]
    nbp2 = o.shape[0]
    for p in range(nbp2):
        o_ref[2 * p] = o[p, 0:c]
        o_ref[2 * p + 1] = o[p, c:2 * c]


def _round_up(v, m):
    return (v + m - 1) // m * m


def kernel(x_nchw, conv_w_oihw, gamma, beta):
    eps = 1e-5
    N, C_in, H, W = x_nchw.shape
    C_out = conv_w_oihw.shape[0]
    HW = H * W
    K = 72
    Lx = (H + 2) * W + 2
    Lx_pad = _round_up(Lx, 128)
    vmem_limit = ((64 << 20) * 3) // 4

    NP = N // 2
    c2 = 2 * C_out
    G = 128
    nbp = 64
    while NP % nbp != 0:
        nbp //= 2
    steps = NP // nbp
    nbp2 = 64
    while NP % nbp2 != 0:
        nbp2 //= 2
    steps2 = NP // nbp2

    xf = x_nchw.astype(jnp.float32)
    x_hp = jnp.pad(xf, ((0, 0), (0, 0), (1, 1), (0, 0)))
    x_flat = x_hp.reshape(N, C_in, (H + 2) * W)
    x_in = jnp.pad(x_flat, ((0, 0), (0, 0), (1, 1 + Lx_pad - Lx)))
    x_in = x_in.reshape(NP, 2, C_in, Lx_pad)

    w_t = jnp.transpose(conv_w_oihw.astype(jnp.float32), (0, 2, 3, 1))
    w_t = w_t.reshape(C_out, 9, C_in)
    w_lo = jnp.pad(w_t, ((0, 0), (0, 0), (0, 5))).reshape(C_out, K)
    w_hi = jnp.pad(w_t, ((0, 0), (0, 0), (4, 1))).reshape(C_out, K)
    w2 = jnp.concatenate([w_lo, w_hi], axis=0)
    g2 = gamma.reshape(C_out, 1).astype(jnp.float32)
    b2 = beta.reshape(C_out, 1).astype(jnp.float32)

    conv_stats = functools.partial(_conv_stats_kernel, nbp=nbp, c2=c2,
                                   W=W, HW=HW, K=K, G=G)
    y16, sums, sqs = pl.pallas_call(
        conv_stats,
        out_shape=(jax.ShapeDtypeStruct((NP, c2, HW), jnp.bfloat16),
                   jax.ShapeDtypeStruct((c2, 1), jnp.float32),
                   jax.ShapeDtypeStruct((c2, 1), jnp.float32)),
        grid=(steps,),
        in_specs=[
            pl.BlockSpec((nbp, 2, C_in, Lx_pad), lambda j: (j, 0, 0, 0)),
            pl.BlockSpec((c2, K), lambda j: (0, 0)),
        ],
        out_specs=(pl.BlockSpec((nbp, c2, HW), lambda j: (j, 0, 0)),
                   pl.BlockSpec((c2, 1), lambda j: (0, 0)),
                   pl.BlockSpec((c2, 1), lambda j: (0, 0))),
        scratch_shapes=[pltpu.VMEM((nbp, 8, Lx_pad), jnp.float32),
                        pltpu.VMEM((nbp, K, HW), jnp.float32)],
        compiler_params=pltpu.CompilerParams(
            dimension_semantics=("arbitrary",),
            vmem_limit_bytes=vmem_limit),
    )(x_in, w2)

    aff = functools.partial(_affine_kernel, m_total=N * H * W, eps=eps,
                            c_out=C_out)
    out_flat = pl.pallas_call(
        aff,
        out_shape=jax.ShapeDtypeStruct((N, C_out, HW), jnp.float32),
        grid=(steps2,),
        in_specs=[
            pl.BlockSpec((nbp2, c2, HW), lambda j: (j, 0, 0)),
            pl.BlockSpec((c2, 1), lambda j: (0, 0)),
            pl.BlockSpec((c2, 1), lambda j: (0, 0)),
            pl.BlockSpec((C_out, 1), lambda j: (0, 0)),
            pl.BlockSpec((C_out, 1), lambda j: (0, 0)),
        ],
        out_specs=pl.BlockSpec((2 * nbp2, C_out, HW), lambda j: (j, 0, 0)),
        compiler_params=pltpu.CompilerParams(
            dimension_semantics=("arbitrary",),
            vmem_limit_bytes=vmem_limit),
    )(y16, sums, sqs, g2, b2)

    return out_flat.reshape(N, C_out, H, W)
